# baseline (speedup 1.0000x reference)
import jax
import jax.numpy as jnp
from jax import lax
from jax.experimental import pallas as pl
from jax.experimental.pallas import tpu as pltpu
from jax.experimental.pallas import tpu_sc as plsc

N_NODES = 10000
N_EDGES = 320000
D = 128

NC = 2
NS = 16

N_PAD = 10240
ROWS_PER_TILE = N_PAD // NS
EDGES_PER_TILE = N_EDGES // (NC * NS)
CHUNK = 128
NFULL = EDGES_PER_TILE // CHUNK
TAIL = EDGES_PER_TILE - NFULL * CHUNK
ACH = 80
NACH = EDGES_PER_TILE // ACH
NGROUPS = (NACH - 1) // 4

_MESH = plsc.VectorSubcoreMesh(core_axis_name="c", subcore_axis_name="s")


def _hist_body(src_hbm, dst_hbm, zeros1_hbm, ones_hbm,
               oa_hbm, ob_hbm, ia_hbm, ib_hbm,
               ohist_sh, ihist_sh, sidx_v, didx_v, ones_v, sema, semb):
    c = lax.axis_index("c")
    s = lax.axis_index("s")
    r0 = s * ROWS_PER_TILE
    ebase = (c * NS + s) * EDGES_PER_TILE
    pltpu.sync_copy(zeros1_hbm, ohist_sh.at[pl.ds(r0, ROWS_PER_TILE)])
    pltpu.sync_copy(zeros1_hbm, ihist_sh.at[pl.ds(r0, ROWS_PER_TILE)])
    pltpu.sync_copy(src_hbm.at[pl.ds(ebase, EDGES_PER_TILE)], sidx_v)
    pltpu.sync_copy(dst_hbm.at[pl.ds(ebase, EDGES_PER_TILE)], didx_v)
    pltpu.sync_copy(ones_hbm, ones_v)
    plsc.subcore_barrier()

    def start(j, n):
        osl = ohist_sh.at[sidx_v.at[pl.ds(j * CHUNK, n)]]
        isl = ihist_sh.at[didx_v.at[pl.ds(j * CHUNK, n)]]
        pltpu.async_copy(ones_v.at[pl.ds(0, n)], osl, sema, add=True)
        pltpu.async_copy(ones_v.at[pl.ds(0, n)], isl, semb, add=True)

    def wait(j, n):
        osl = ohist_sh.at[sidx_v.at[pl.ds(j * CHUNK, n)]]
        isl = ihist_sh.at[didx_v.at[pl.ds(j * CHUNK, n)]]
        pltpu.make_async_copy(ones_v.at[pl.ds(0, n)], osl, sema).wait()
        pltpu.make_async_copy(ones_v.at[pl.ds(0, n)], isl, semb).wait()

    start(0, CHUNK)

    def body(j, carry):
        wait(j - 1, CHUNK)

        @pl.when(j < NFULL)
        def _():
            start(j, CHUNK)

        return carry

    lax.fori_loop(1, NFULL + 1, body, 0)
    start(NFULL, TAIL)
    wait(NFULL, TAIL)
    plsc.subcore_barrier()

    @pl.when(c == 0)
    def _():
        pltpu.sync_copy(ohist_sh.at[pl.ds(r0, ROWS_PER_TILE)],
                        oa_hbm.at[pl.ds(r0, ROWS_PER_TILE)])
        pltpu.sync_copy(ihist_sh.at[pl.ds(r0, ROWS_PER_TILE)],
                        ia_hbm.at[pl.ds(r0, ROWS_PER_TILE)])

    @pl.when(c == 1)
    def _():
        pltpu.sync_copy(ohist_sh.at[pl.ds(r0, ROWS_PER_TILE)],
                        ob_hbm.at[pl.ds(r0, ROWS_PER_TILE)])
        pltpu.sync_copy(ihist_sh.at[pl.ds(r0, ROWS_PER_TILE)],
                        ib_hbm.at[pl.ds(r0, ROWS_PER_TILE)])


_deg1 = jax.ShapeDtypeStruct((N_PAD,), jnp.float32)
_hist = pl.kernel(
    _hist_body,
    out_type=(_deg1, _deg1, _deg1, _deg1),
    mesh=_MESH,
    scratch_types=[
        pltpu.VMEM_SHARED((N_PAD,), jnp.float32),
        pltpu.VMEM_SHARED((N_PAD,), jnp.float32),
        pltpu.VMEM((EDGES_PER_TILE,), jnp.int32),
        pltpu.VMEM((EDGES_PER_TILE,), jnp.int32),
        pltpu.VMEM((CHUNK,), jnp.float32),
        pltpu.SemaphoreType.DMA,
        pltpu.SemaphoreType.DMA,
    ],
)


def _agg_body(msg_hbm, src_hbm, dst_hbm, zeros_hbm,
              agg_hbm,
              agg_sh,
              sidx0, didx0, sidx1, didx1, sidx2, didx2, sidx3, didx3,
              rows0, rows1, rows2, rows3,
              semi0, semi1, semi2, semi3, semg0, semg1, semg2, semg3):
    c = lax.axis_index("c")
    s = lax.axis_index("s")
    r0 = s * ROWS_PER_TILE
    ebase = (c * NS + s) * EDGES_PER_TILE
    pltpu.sync_copy(zeros_hbm, agg_sh.at[pl.ds(r0, ROWS_PER_TILE)])
    plsc.subcore_barrier()

    sidx = (sidx0, sidx1, sidx2, sidx3)
    didx = (didx0, didx1, didx2, didx3)
    rows = (rows0, rows1, rows2, rows3)
    semi = (semi0, semi1, semi2, semi3)
    semg = (semg0, semg1, semg2, semg3)

    def start_idx(j, b):
        pltpu.async_copy(src_hbm.at[pl.ds(ebase + j * ACH, ACH)],
                         sidx[b], semi[b])
        pltpu.async_copy(dst_hbm.at[pl.ds(ebase + j * ACH, ACH)],
                         didx[b], semi[b])

    def wait_idx(j, b):
        pltpu.make_async_copy(src_hbm.at[pl.ds(ebase + j * ACH, ACH)],
                              sidx[b], semi[b]).wait()
        pltpu.make_async_copy(dst_hbm.at[pl.ds(ebase + j * ACH, ACH)],
                              didx[b], semi[b]).wait()

    def start_gather(b):
        pltpu.async_copy(msg_hbm.at[sidx[b]], rows[b], semg[b])

    def wait_gather(b):
        pltpu.make_async_copy(msg_hbm.at[sidx[b]], rows[b], semg[b]).wait()

    for b in range(4):
        start_idx(b, b)
    for b in range(4):
        wait_idx(b, b)
        start_gather(b)

    def group(g, carry):
        for b in range(4):
            wait_gather(b)
            pltpu.sync_copy(rows[b], agg_sh.at[didx[b]], add=True)
            nxt = 4 * g + 4 + b

            @pl.when(nxt < NACH)
            def _():
                start_idx(nxt, b)
                wait_idx(nxt, b)
                start_gather(b)

        return carry

    lax.fori_loop(0, NGROUPS, group, 0)
    wait_gather(0)
    pltpu.sync_copy(rows0, agg_sh.at[didx0], add=True)
    plsc.subcore_barrier()
    pltpu.sync_copy(agg_sh.at[pl.ds(r0, ROWS_PER_TILE)],
                    agg_hbm.at[c, pl.ds(r0, ROWS_PER_TILE)])


_agg = pl.kernel(
    _agg_body,
    out_type=jax.ShapeDtypeStruct((NC, N_PAD, D), jnp.float32),
    mesh=_MESH,
    scratch_types=(
        [pltpu.VMEM_SHARED((N_PAD, D), jnp.float32)]
        + [pltpu.VMEM((ACH,), jnp.int32) for _ in range(8)]
        + [pltpu.VMEM((ACH, D), jnp.float32) for _ in range(4)]
        + [pltpu.SemaphoreType.DMA for _ in range(8)]
    ),
)


BLK = 2000


def _msg_body(x_ref, outdeg_ref, msg_ref):
    deg = jnp.maximum(outdeg_ref[...], 1.0)
    msg_ref[...] = x_ref[...] * lax.rsqrt(deg)


def _msg(x, outdeg):
    return pl.pallas_call(
        _msg_body,
        out_shape=jax.ShapeDtypeStruct((N_NODES, D), jnp.float32),
        grid=(N_NODES // BLK,),
        in_specs=[
            pl.BlockSpec((BLK, D), lambda i: (i, 0)),
            pl.BlockSpec((BLK, 1), lambda i: (i, 0)),
        ],
        out_specs=pl.BlockSpec((BLK, D), lambda i: (i, 0)),
    )(x, outdeg)


def _final_body(agg_ref, indeg_ref, w_ref, b_ref, out_ref):
    a = agg_ref[0] + agg_ref[1]
    deg = jnp.maximum(indeg_ref[...], 1.0)
    h = a * lax.rsqrt(deg)
    out_ref[...] = jnp.dot(h, w_ref[...],
                           preferred_element_type=jnp.float32) + b_ref[...]


def _final(agg, indeg, W, b2):
    return pl.pallas_call(
        _final_body,
        out_shape=jax.ShapeDtypeStruct((N_NODES, D), jnp.float32),
        grid=(N_NODES // BLK,),
        in_specs=[
            pl.BlockSpec((NC, BLK, D), lambda i: (0, i, 0)),
            pl.BlockSpec((BLK, 1), lambda i: (i, 0)),
            pl.BlockSpec((D, D), lambda i: (0, 0)),
            pl.BlockSpec((1, D), lambda i: (0, 0)),
        ],
        out_specs=pl.BlockSpec((BLK, D), lambda i: (i, 0)),
    )(agg, indeg, W, b2)


def kernel(inputs, edge_index, W, b):
    x = inputs
    src = edge_index[0]
    dst = edge_index[1]
    zeros1 = jnp.zeros((ROWS_PER_TILE,), jnp.float32)
    ones1 = jnp.ones((CHUNK,), jnp.float32)
    oa, ob, ia, ib = _hist(src, dst, zeros1, ones1)
    outdeg_col = (oa + ob)[:N_NODES].reshape(N_NODES, 1)
    indeg_col = (ia + ib)[:N_NODES].reshape(N_NODES, 1)
    msg = _msg(x, outdeg_col)
    zerosN = jnp.zeros((ROWS_PER_TILE, D), jnp.float32)
    agg = _agg(msg, src, dst, zerosN)
    return _final(agg, indeg_col, W, b.reshape(1, D))


# speedup vs baseline: 1.0251x; 1.0251x over previous
import jax
import jax.numpy as jnp
from jax import lax
from jax.experimental import pallas as pl
from jax.experimental.pallas import tpu as pltpu
from jax.experimental.pallas import tpu_sc as plsc

N_NODES = 10000
N_EDGES = 320000
D = 128

NC = 2
NS = 16

N_PAD = 10240
ROWS_PER_TILE = N_PAD // NS
EDGES_PER_TILE = N_EDGES // (NC * NS)
CHUNK = 128
NFULL = EDGES_PER_TILE // CHUNK
NPAIRS = NFULL // 2
TAIL = EDGES_PER_TILE - NFULL * CHUNK

_MESH = plsc.VectorSubcoreMesh(core_axis_name="c", subcore_axis_name="s")


def _hist_body(src_hbm, dst_hbm, zeros1_hbm, ones_hbm,
               oa_hbm, ob_hbm, ia_hbm, ib_hbm,
               ohist_sh, ihist_sh, sidx_v, didx_v, ones_v, sema, semb):
    c = lax.axis_index("c")
    s = lax.axis_index("s")
    r0 = s * ROWS_PER_TILE
    ebase = (c * NS + s) * EDGES_PER_TILE
    pltpu.sync_copy(zeros1_hbm.at[pl.ds(r0, ROWS_PER_TILE)],
                    ohist_sh.at[pl.ds(r0, ROWS_PER_TILE)])
    pltpu.sync_copy(zeros1_hbm.at[pl.ds(r0, ROWS_PER_TILE)],
                    ihist_sh.at[pl.ds(r0, ROWS_PER_TILE)])
    pltpu.sync_copy(src_hbm.at[pl.ds(ebase, EDGES_PER_TILE)], sidx_v)
    pltpu.sync_copy(dst_hbm.at[pl.ds(ebase, EDGES_PER_TILE)], didx_v)
    pltpu.sync_copy(ones_hbm, ones_v)
    plsc.subcore_barrier()

    def start(j, n):
        osl = ohist_sh.at[sidx_v.at[pl.ds(j * CHUNK, n)]]
        isl = ihist_sh.at[didx_v.at[pl.ds(j * CHUNK, n)]]
        pltpu.async_copy(ones_v.at[pl.ds(0, n)], osl, sema, add=True)
        pltpu.async_copy(ones_v.at[pl.ds(0, n)], isl, semb, add=True)

    def wait(j, n):
        osl = ohist_sh.at[sidx_v.at[pl.ds(j * CHUNK, n)]]
        isl = ihist_sh.at[didx_v.at[pl.ds(j * CHUNK, n)]]
        pltpu.make_async_copy(ones_v.at[pl.ds(0, n)], osl, sema).wait()
        pltpu.make_async_copy(ones_v.at[pl.ds(0, n)], isl, semb).wait()

    start(0, CHUNK)

    def body(j, carry):
        wait(j - 1, CHUNK)

        @pl.when(j < NFULL)
        def _():
            start(j, CHUNK)

        return carry

    lax.fori_loop(1, NFULL + 1, body, 0)
    start(NFULL, TAIL)
    wait(NFULL, TAIL)
    plsc.subcore_barrier()

    @pl.when(c == 0)
    def _():
        pltpu.sync_copy(ohist_sh.at[pl.ds(r0, ROWS_PER_TILE)],
                        oa_hbm.at[pl.ds(r0, ROWS_PER_TILE)])
        pltpu.sync_copy(ihist_sh.at[pl.ds(r0, ROWS_PER_TILE)],
                        ia_hbm.at[pl.ds(r0, ROWS_PER_TILE)])

    @pl.when(c == 1)
    def _():
        pltpu.sync_copy(ohist_sh.at[pl.ds(r0, ROWS_PER_TILE)],
                        ob_hbm.at[pl.ds(r0, ROWS_PER_TILE)])
        pltpu.sync_copy(ihist_sh.at[pl.ds(r0, ROWS_PER_TILE)],
                        ib_hbm.at[pl.ds(r0, ROWS_PER_TILE)])


_deg1 = jax.ShapeDtypeStruct((N_PAD,), jnp.float32)
_hist = pl.kernel(
    _hist_body,
    out_type=(_deg1, _deg1, _deg1, _deg1),
    mesh=_MESH,
    scratch_types=[
        pltpu.VMEM_SHARED((N_PAD,), jnp.float32),
        pltpu.VMEM_SHARED((N_PAD,), jnp.float32),
        pltpu.VMEM((EDGES_PER_TILE,), jnp.int32),
        pltpu.VMEM((EDGES_PER_TILE,), jnp.int32),
        pltpu.VMEM((CHUNK,), jnp.float32),
        pltpu.SemaphoreType.DMA,
        pltpu.SemaphoreType.DMA,
    ],
)


def _agg_body(msg_hbm, src_hbm, dst_hbm, zeros_hbm,
              agg_hbm,
              agg_sh, sidx0, didx0, sidx1, didx1, rows0, rows1,
              semi0, semi1, semg0, semg1):
    c = lax.axis_index("c")
    s = lax.axis_index("s")
    r0 = s * ROWS_PER_TILE
    ebase = (c * NS + s) * EDGES_PER_TILE
    pltpu.sync_copy(zeros_hbm.at[pl.ds(r0, ROWS_PER_TILE)],
                    agg_sh.at[pl.ds(r0, ROWS_PER_TILE)])
    plsc.subcore_barrier()

    def start_idx(j, sv, dv, sem):
        pltpu.async_copy(src_hbm.at[pl.ds(ebase + j * CHUNK, CHUNK)], sv, sem)
        pltpu.async_copy(dst_hbm.at[pl.ds(ebase + j * CHUNK, CHUNK)], dv, sem)

    def wait_idx(j, sv, dv, sem):
        pltpu.make_async_copy(src_hbm.at[pl.ds(ebase + j * CHUNK, CHUNK)],
                              sv, sem).wait()
        pltpu.make_async_copy(dst_hbm.at[pl.ds(ebase + j * CHUNK, CHUNK)],
                              dv, sem).wait()

    def scatter(rows, dv):
        pltpu.sync_copy(rows, agg_sh.at[dv], add=True)

    start_idx(0, sidx0, didx0, semi0)
    start_idx(1, sidx1, didx1, semi1)
    wait_idx(0, sidx0, didx0, semi0)
    pltpu.async_copy(msg_hbm.at[sidx0], rows0, semg0)
    wait_idx(1, sidx1, didx1, semi1)

    def pair(i, carry):
        a = 2 * i
        pltpu.make_async_copy(msg_hbm.at[sidx0], rows0, semg0).wait()
        pltpu.async_copy(msg_hbm.at[sidx1], rows1, semg1)
        scatter(rows0, didx0)

        @pl.when(i < NPAIRS - 1)
        def _():
            start_idx(a + 2, sidx0, didx0, semi0)

        pltpu.make_async_copy(msg_hbm.at[sidx1], rows1, semg1).wait()

        @pl.when(i < NPAIRS - 1)
        def _():
            wait_idx(a + 2, sidx0, didx0, semi0)
            pltpu.async_copy(msg_hbm.at[sidx0], rows0, semg0)

        scatter(rows1, didx1)

        @pl.when(i < NPAIRS - 1)
        def _():
            start_idx(a + 3, sidx1, didx1, semi1)
            wait_idx(a + 3, sidx1, didx1, semi1)

        return carry

    lax.fori_loop(0, NPAIRS, pair, 0)

    tb = ebase + NFULL * CHUNK
    pltpu.sync_copy(src_hbm.at[pl.ds(tb, TAIL)], sidx0.at[pl.ds(0, TAIL)])
    pltpu.sync_copy(dst_hbm.at[pl.ds(tb, TAIL)], didx0.at[pl.ds(0, TAIL)])
    pltpu.sync_copy(msg_hbm.at[sidx0.at[pl.ds(0, TAIL)]],
                    rows0.at[pl.ds(0, TAIL)])
    pltpu.sync_copy(rows0.at[pl.ds(0, TAIL)],
                    agg_sh.at[didx0.at[pl.ds(0, TAIL)]], add=True)
    plsc.subcore_barrier()
    pltpu.sync_copy(agg_sh.at[pl.ds(r0, ROWS_PER_TILE)],
                    agg_hbm.at[c, pl.ds(r0, ROWS_PER_TILE)])


_agg = pl.kernel(
    _agg_body,
    out_type=jax.ShapeDtypeStruct((NC, N_PAD, D), jnp.float32),
    mesh=_MESH,
    scratch_types=[
        pltpu.VMEM_SHARED((N_PAD, D), jnp.float32),
        pltpu.VMEM((CHUNK,), jnp.int32),
        pltpu.VMEM((CHUNK,), jnp.int32),
        pltpu.VMEM((CHUNK,), jnp.int32),
        pltpu.VMEM((CHUNK,), jnp.int32),
        pltpu.VMEM((CHUNK, D), jnp.float32),
        pltpu.VMEM((CHUNK, D), jnp.float32),
        pltpu.SemaphoreType.DMA,
        pltpu.SemaphoreType.DMA,
        pltpu.SemaphoreType.DMA,
        pltpu.SemaphoreType.DMA,
    ],
)


BLK = 2000


def _xw_body(x_ref, w_ref, y_ref):
    y_ref[...] = jnp.dot(x_ref[...], w_ref[...],
                         preferred_element_type=jnp.float32)


def _xw(x, W):
    return pl.pallas_call(
        _xw_body,
        out_shape=jax.ShapeDtypeStruct((N_NODES, D), jnp.float32),
        grid=(N_NODES // BLK,),
        in_specs=[
            pl.BlockSpec((BLK, D), lambda i: (i, 0)),
            pl.BlockSpec((D, D), lambda i: (0, 0)),
        ],
        out_specs=pl.BlockSpec((BLK, D), lambda i: (i, 0)),
    )(x, W)


def _msg_body(x_ref, outdeg_ref, msg_ref):
    deg = jnp.maximum(outdeg_ref[...], 1.0)
    msg_ref[...] = x_ref[...] * lax.rsqrt(deg)


def _msg(x, outdeg):
    return pl.pallas_call(
        _msg_body,
        out_shape=jax.ShapeDtypeStruct((N_NODES, D), jnp.float32),
        grid=(N_NODES // BLK,),
        in_specs=[
            pl.BlockSpec((BLK, D), lambda i: (i, 0)),
            pl.BlockSpec((BLK, 1), lambda i: (i, 0)),
        ],
        out_specs=pl.BlockSpec((BLK, D), lambda i: (i, 0)),
    )(x, outdeg)


def _final_body(agg_ref, indeg_ref, b_ref, out_ref):
    a = agg_ref[0] + agg_ref[1]
    deg = jnp.maximum(indeg_ref[...], 1.0)
    out_ref[...] = a * lax.rsqrt(deg) + b_ref[...]


def _final(agg, indeg, b2):
    return pl.pallas_call(
        _final_body,
        out_shape=jax.ShapeDtypeStruct((N_NODES, D), jnp.float32),
        grid=(N_NODES // BLK,),
        in_specs=[
            pl.BlockSpec((NC, BLK, D), lambda i: (0, i, 0)),
            pl.BlockSpec((BLK, 1), lambda i: (i, 0)),
            pl.BlockSpec((1, D), lambda i: (0, 0)),
        ],
        out_specs=pl.BlockSpec((BLK, D), lambda i: (i, 0)),
    )(agg, indeg, b2)


def kernel(inputs, edge_index, W, b):
    x = inputs
    src = edge_index[0]
    dst = edge_index[1]
    zeros1 = jnp.zeros((N_PAD,), jnp.float32)
    ones1 = jnp.ones((CHUNK,), jnp.float32)
    oa, ob, ia, ib = _hist(src, dst, zeros1, ones1)
    y = _xw(x, W)
    outdeg_col = (oa + ob)[:N_NODES].reshape(N_NODES, 1)
    indeg_col = (ia + ib)[:N_NODES].reshape(N_NODES, 1)
    msg = _msg(y, outdeg_col)
    zerosN = jnp.zeros((N_PAD, D), jnp.float32)
    agg = _agg(msg, src, dst, zerosN)
    return _final(agg, indeg_col, b.reshape(1, D))


# speedup vs baseline: 1.0524x; 1.0266x over previous
import jax
import jax.numpy as jnp
from jax import lax
from jax.experimental import pallas as pl
from jax.experimental.pallas import tpu as pltpu
from jax.experimental.pallas import tpu_sc as plsc

N_NODES = 10000
N_EDGES = 320000
D = 128

NC = 2
NS = 16

N_PAD = 10240
ROWS_PER_TILE = N_PAD // NS
EDGES_PER_TILE = N_EDGES // (NC * NS)
CHUNK = 128
NFULL = EDGES_PER_TILE // CHUNK
NPAIRS = NFULL // 2
TAIL = EDGES_PER_TILE - NFULL * CHUNK

_MESH = plsc.VectorSubcoreMesh(core_axis_name="c", subcore_axis_name="s")


def _hist_body(src_hbm, dst_hbm, zeros1_hbm, ones_hbm,
               oa_hbm, ob_hbm, ia_hbm, ib_hbm,
               ohist_sh, ihist_sh, sidx_v, didx_v, ones_v, sema, semb):
    c = lax.axis_index("c")
    s = lax.axis_index("s")
    r0 = s * ROWS_PER_TILE
    ebase = (c * NS + s) * EDGES_PER_TILE
    pltpu.sync_copy(zeros1_hbm, ohist_sh.at[pl.ds(r0, ROWS_PER_TILE)])
    pltpu.sync_copy(zeros1_hbm, ihist_sh.at[pl.ds(r0, ROWS_PER_TILE)])
    pltpu.sync_copy(src_hbm.at[pl.ds(ebase, EDGES_PER_TILE)], sidx_v)
    pltpu.sync_copy(dst_hbm.at[pl.ds(ebase, EDGES_PER_TILE)], didx_v)
    pltpu.sync_copy(ones_hbm, ones_v)
    plsc.subcore_barrier()

    def start(j, n):
        osl = ohist_sh.at[sidx_v.at[pl.ds(j * CHUNK, n)]]
        isl = ihist_sh.at[didx_v.at[pl.ds(j * CHUNK, n)]]
        pltpu.async_copy(ones_v.at[pl.ds(0, n)], osl, sema, add=True)
        pltpu.async_copy(ones_v.at[pl.ds(0, n)], isl, semb, add=True)

    def wait(j, n):
        osl = ohist_sh.at[sidx_v.at[pl.ds(j * CHUNK, n)]]
        isl = ihist_sh.at[didx_v.at[pl.ds(j * CHUNK, n)]]
        pltpu.make_async_copy(ones_v.at[pl.ds(0, n)], osl, sema).wait()
        pltpu.make_async_copy(ones_v.at[pl.ds(0, n)], isl, semb).wait()

    start(0, CHUNK)
    start(1, CHUNK)
    start(2, CHUNK)
    start(3, CHUNK)

    def body(j, carry):
        wait(j - 4, CHUNK)

        @pl.when(j < NFULL)
        def _():
            start(j, CHUNK)

        return carry

    lax.fori_loop(4, NFULL + 4, body, 0)
    start(NFULL, TAIL)
    wait(NFULL, TAIL)
    plsc.subcore_barrier()

    @pl.when(c == 0)
    def _():
        pltpu.sync_copy(ohist_sh.at[pl.ds(r0, ROWS_PER_TILE)],
                        oa_hbm.at[pl.ds(r0, ROWS_PER_TILE)])
        pltpu.sync_copy(ihist_sh.at[pl.ds(r0, ROWS_PER_TILE)],
                        ia_hbm.at[pl.ds(r0, ROWS_PER_TILE)])

    @pl.when(c == 1)
    def _():
        pltpu.sync_copy(ohist_sh.at[pl.ds(r0, ROWS_PER_TILE)],
                        ob_hbm.at[pl.ds(r0, ROWS_PER_TILE)])
        pltpu.sync_copy(ihist_sh.at[pl.ds(r0, ROWS_PER_TILE)],
                        ib_hbm.at[pl.ds(r0, ROWS_PER_TILE)])


_deg1 = jax.ShapeDtypeStruct((N_PAD,), jnp.float32)
_hist = pl.kernel(
    _hist_body,
    out_type=(_deg1, _deg1, _deg1, _deg1),
    mesh=_MESH,
    scratch_types=[
        pltpu.VMEM_SHARED((N_PAD,), jnp.float32),
        pltpu.VMEM_SHARED((N_PAD,), jnp.float32),
        pltpu.VMEM((EDGES_PER_TILE,), jnp.int32),
        pltpu.VMEM((EDGES_PER_TILE,), jnp.int32),
        pltpu.VMEM((CHUNK,), jnp.float32),
        pltpu.SemaphoreType.DMA,
        pltpu.SemaphoreType.DMA,
    ],
)


def _agg_body(msg_hbm, src_hbm, dst_hbm, zeros_hbm,
              agg_hbm,
              agg_sh, sidx0, didx0, sidx1, didx1, rows0, rows1,
              semi0, semi1, semg0, semg1):
    c = lax.axis_index("c")
    s = lax.axis_index("s")
    r0 = s * ROWS_PER_TILE
    ebase = (c * NS + s) * EDGES_PER_TILE
    pltpu.sync_copy(zeros_hbm, agg_sh.at[pl.ds(r0, ROWS_PER_TILE)])
    plsc.subcore_barrier()

    def start_idx(j, sv, dv, sem):
        pltpu.async_copy(src_hbm.at[pl.ds(ebase + j * CHUNK, CHUNK)], sv, sem)
        pltpu.async_copy(dst_hbm.at[pl.ds(ebase + j * CHUNK, CHUNK)], dv, sem)

    def wait_idx(j, sv, dv, sem):
        pltpu.make_async_copy(src_hbm.at[pl.ds(ebase + j * CHUNK, CHUNK)],
                              sv, sem).wait()
        pltpu.make_async_copy(dst_hbm.at[pl.ds(ebase + j * CHUNK, CHUNK)],
                              dv, sem).wait()

    def scatter(rows, dv):
        pltpu.sync_copy(rows, agg_sh.at[dv], add=True)

    start_idx(0, sidx0, didx0, semi0)
    start_idx(1, sidx1, didx1, semi1)
    wait_idx(0, sidx0, didx0, semi0)
    pltpu.async_copy(msg_hbm.at[sidx0], rows0, semg0)
    wait_idx(1, sidx1, didx1, semi1)

    def pair(i, carry):
        a = 2 * i
        pltpu.make_async_copy(msg_hbm.at[sidx0], rows0, semg0).wait()
        pltpu.async_copy(msg_hbm.at[sidx1], rows1, semg1)
        scatter(rows0, didx0)

        @pl.when(i < NPAIRS - 1)
        def _():
            start_idx(a + 2, sidx0, didx0, semi0)

        pltpu.make_async_copy(msg_hbm.at[sidx1], rows1, semg1).wait()

        @pl.when(i < NPAIRS - 1)
        def _():
            wait_idx(a + 2, sidx0, didx0, semi0)
            pltpu.async_copy(msg_hbm.at[sidx0], rows0, semg0)

        scatter(rows1, didx1)

        @pl.when(i < NPAIRS - 1)
        def _():
            start_idx(a + 3, sidx1, didx1, semi1)
            wait_idx(a + 3, sidx1, didx1, semi1)

        return carry

    lax.fori_loop(0, NPAIRS, pair, 0)

    tb = ebase + NFULL * CHUNK
    pltpu.sync_copy(src_hbm.at[pl.ds(tb, TAIL)], sidx0.at[pl.ds(0, TAIL)])
    pltpu.sync_copy(dst_hbm.at[pl.ds(tb, TAIL)], didx0.at[pl.ds(0, TAIL)])
    pltpu.sync_copy(msg_hbm.at[sidx0.at[pl.ds(0, TAIL)]],
                    rows0.at[pl.ds(0, TAIL)])
    pltpu.sync_copy(rows0.at[pl.ds(0, TAIL)],
                    agg_sh.at[didx0.at[pl.ds(0, TAIL)]], add=True)
    plsc.subcore_barrier()
    pltpu.sync_copy(agg_sh.at[pl.ds(r0, ROWS_PER_TILE)],
                    agg_hbm.at[c, pl.ds(r0, ROWS_PER_TILE)])


_agg = pl.kernel(
    _agg_body,
    out_type=jax.ShapeDtypeStruct((NC, N_PAD, D), jnp.float32),
    mesh=_MESH,
    scratch_types=[
        pltpu.VMEM_SHARED((N_PAD, D), jnp.float32),
        pltpu.VMEM((CHUNK,), jnp.int32),
        pltpu.VMEM((CHUNK,), jnp.int32),
        pltpu.VMEM((CHUNK,), jnp.int32),
        pltpu.VMEM((CHUNK,), jnp.int32),
        pltpu.VMEM((CHUNK, D), jnp.float32),
        pltpu.VMEM((CHUNK, D), jnp.float32),
        pltpu.SemaphoreType.DMA,
        pltpu.SemaphoreType.DMA,
        pltpu.SemaphoreType.DMA,
        pltpu.SemaphoreType.DMA,
    ],
)


BLK = 2000


def _msg_body(x_ref, outdeg_ref, msg_ref):
    deg = jnp.maximum(outdeg_ref[...], 1.0)
    msg_ref[...] = x_ref[...] * lax.rsqrt(deg)


def _msg(x, outdeg):
    return pl.pallas_call(
        _msg_body,
        out_shape=jax.ShapeDtypeStruct((N_NODES, D), jnp.float32),
        grid=(N_NODES // BLK,),
        in_specs=[
            pl.BlockSpec((BLK, D), lambda i: (i, 0)),
            pl.BlockSpec((BLK, 1), lambda i: (i, 0)),
        ],
        out_specs=pl.BlockSpec((BLK, D), lambda i: (i, 0)),
    )(x, outdeg)


def _final_body(agg_ref, indeg_ref, w_ref, b_ref, out_ref):
    a = agg_ref[0] + agg_ref[1]
    deg = jnp.maximum(indeg_ref[...], 1.0)
    h = a * lax.rsqrt(deg)
    out_ref[...] = jnp.dot(h, w_ref[...],
                           preferred_element_type=jnp.float32) + b_ref[...]


def _final(agg, indeg, W, b2):
    return pl.pallas_call(
        _final_body,
        out_shape=jax.ShapeDtypeStruct((N_NODES, D), jnp.float32),
        grid=(N_NODES // BLK,),
        in_specs=[
            pl.BlockSpec((NC, BLK, D), lambda i: (0, i, 0)),
            pl.BlockSpec((BLK, 1), lambda i: (i, 0)),
            pl.BlockSpec((D, D), lambda i: (0, 0)),
            pl.BlockSpec((1, D), lambda i: (0, 0)),
        ],
        out_specs=pl.BlockSpec((BLK, D), lambda i: (i, 0)),
    )(agg, indeg, W, b2)


def kernel(inputs, edge_index, W, b):
    x = inputs
    src = edge_index[0]
    dst = edge_index[1]
    zeros1 = jnp.zeros((ROWS_PER_TILE,), jnp.float32)
    ones1 = jnp.ones((CHUNK,), jnp.float32)
    oa, ob, ia, ib = _hist(src, dst, zeros1, ones1)
    outdeg_col = (oa + ob)[:N_NODES].reshape(N_NODES, 1)
    indeg_col = (ia + ib)[:N_NODES].reshape(N_NODES, 1)
    msg = _msg(x, outdeg_col)
    zerosN = jnp.zeros((ROWS_PER_TILE, D), jnp.float32)
    agg = _agg(msg, src, dst, zerosN)
    return _final(agg, indeg_col, W, b.reshape(1, D))
